# baseline (speedup 1.0000x reference)
import functools

import jax
import jax.numpy as jnp
from jax import lax
from jax.experimental import pallas as pl
from jax.experimental.pallas import tpu as pltpu
from jax.experimental.pallas import tpu_sc as plsc

D_MODEL = 32
NUM_WORKERS = 32
BLOCK = 256
HI_ROWS = 1024
PACK = 128 // D_MODEL
TROWS = HI_ROWS // PACK

_DYN_GATHER_DNUMS = lax.GatherDimensionNumbers(
    offset_dims=(), collapsed_slice_dims=(0,), start_index_map=(0,)
)


def _lane_perm(a, idx):
    return lax.gather(
        a, idx[:, None], _DYN_GATHER_DNUMS, slice_sizes=(1,),
        mode=lax.GatherScatterMode.PROMISE_IN_BOUNDS,
    )


def _emb_body(n_blocks, w_hbm, idx_hbm, hidx_hbm, out_hbm,
              tlo, tlo_s, thi, hidx_v, idx0, idx1, obuf0, obuf1,
              tsem, isem, osem):
    cid = lax.axis_index("c")
    sid = lax.axis_index("s")
    wid = sid * 2 + cid
    base4 = wid * (n_blocks * (BLOCK // PACK))
    iota16 = lax.iota(jnp.int32, 16)
    swap_idx = lax.bitwise_xor(iota16, 1)
    odd_idx = lax.bitwise_or(iota16, 1)
    even_idx = lax.bitwise_and(iota16, ~1)
    sgn = jnp.where(lax.bitwise_and(iota16, 1) == 0, 1.0, -1.0).astype(jnp.float32)

    pltpu.sync_copy(hidx_hbm, hidx_v)
    for r in range(PACK):
        for j in range(2):
            pltpu.async_copy(
                w_hbm.at[hidx_v.at[2 * r + j]],
                tlo_s.at[pl.ds(j * 128, 128)], tsem)
        for j in range(2):
            pltpu.make_async_copy(
                w_hbm.at[hidx_v.at[2 * r + j]],
                tlo_s.at[pl.ds(j * 128, 128)], tsem).wait()

        def compact(k, carry, r=r):
            korig = r * 256 + k
            for h in (0, 16):
                thi[korig >> 2, pl.ds(lax.bitwise_and(k, 3) * D_MODEL + h, 16)] = (
                    tlo_s[k, pl.ds(h, 16)])
            return carry

        lax.fori_loop(0, 256, compact, 0)

    pltpu.sync_copy(w_hbm.at[pl.ds(0, TROWS)], tlo)

    def mk_swapped(rr, carry):
        row = lax.shift_right_logical(rr, 2)
        col = lax.bitwise_and(rr, 3) * D_MODEL
        for h in (0, 16):
            a = tlo[row, pl.ds(col + h, 16)]
            tlo_s[row, pl.ds(col + h, 16)] = _lane_perm(a, swap_idx) * sgn
        return carry

    lax.fori_loop(0, HI_ROWS, mk_swapped, 0)

    def compute_block(idx_v, obuf):
        @plsc.parallel_loop(0, BLOCK // 16, step=1)
        def _(i):
            p16 = idx_v[pl.ds(i * 16, 16)]
            for u in range(16):
                p = p16[u]
                hi = lax.shift_right_logical(p, 10)
                lo = lax.bitwise_and(p, 1023)
                row_lo = lax.shift_right_logical(lo, 2)
                col_lo = lax.bitwise_and(lo, 3) * D_MODEL
                row_hi = lax.shift_right_logical(hi, 2)
                col_hi = lax.bitwise_and(hi, 3) * D_MODEL
                for h in (0, 16):
                    a = tlo[row_lo, pl.ds(col_lo + h, 16)]
                    a_s = tlo_s[row_lo, pl.ds(col_lo + h, 16)]
                    b = thi[row_hi, pl.ds(col_hi + h, 16)]
                    x = _lane_perm(b, odd_idx)
                    v = _lane_perm(b, even_idx)
                    obuf[i * 4 + u // 4, pl.ds((u % 4) * D_MODEL + h, 16)] = (
                        a * x + a_s * v)

    def load_idx(b, idx_v):
        pltpu.async_copy(idx_hbm.at[wid, pl.ds(b * BLOCK, BLOCK)], idx_v, isem)

    def wait_idx(b, idx_v):
        pltpu.make_async_copy(
            idx_hbm.at[wid, pl.ds(b * BLOCK, BLOCK)], idx_v, isem).wait()

    def process(b, idx_v, obuf):
        wait_idx(b, idx_v)

        @pl.when(b >= 2)
        def _():
            pltpu.make_async_copy(
                obuf, out_hbm.at[pl.ds(base4, BLOCK // PACK)], osem).wait()

        compute_block(idx_v, obuf)
        pltpu.async_copy(
            obuf, out_hbm.at[pl.ds(base4 + b * (BLOCK // PACK), BLOCK // PACK)],
            osem)

        @pl.when(b + 2 < n_blocks)
        def _():
            load_idx(b + 2, idx_v)

    load_idx(0, idx0)
    load_idx(1, idx1)

    def body(k, carry):
        process(2 * k, idx0, obuf0)
        process(2 * k + 1, idx1, obuf1)
        return carry

    lax.fori_loop(0, n_blocks // 2, body, 0)
    for obuf in (obuf0, obuf1):
        pltpu.make_async_copy(
            obuf, out_hbm.at[pl.ds(base4, BLOCK // PACK)], osem).wait()


def kernel(x, w):
    batch, seq = x.shape
    n_total = batch * seq
    n_per_worker = n_total // NUM_WORKERS
    n_blocks = n_per_worker // BLOCK
    c_in = w.shape[0]
    w2 = w.reshape(c_in // PACK, 128)
    idx2 = x.reshape(NUM_WORKERS, n_per_worker)
    hidx = jnp.minimum(
        jnp.arange(HI_ROWS, dtype=jnp.int32) * (HI_ROWS // PACK),
        (c_in - 1) // HI_ROWS * (HI_ROWS // PACK),
    ).reshape(HI_ROWS // 128, 128)

    mesh = plsc.VectorSubcoreMesh(core_axis_name="c", subcore_axis_name="s")
    emb = functools.partial(
        pl.kernel,
        out_type=jax.ShapeDtypeStruct((n_total // PACK, 128), jnp.float32),
        mesh=mesh,
        scratch_types=[
            pltpu.VMEM((TROWS, 128), jnp.float32),
            pltpu.VMEM((TROWS, 128), jnp.float32),
            pltpu.VMEM((TROWS, 128), jnp.float32),
            pltpu.VMEM((HI_ROWS // 128, 128), jnp.int32),
            pltpu.VMEM((BLOCK,), jnp.int32),
            pltpu.VMEM((BLOCK,), jnp.int32),
            pltpu.VMEM((BLOCK // PACK, 128), jnp.float32),
            pltpu.VMEM((BLOCK // PACK, 128), jnp.float32),
            pltpu.SemaphoreType.DMA,
            pltpu.SemaphoreType.DMA,
            pltpu.SemaphoreType.DMA,
        ],
        compiler_params=pltpu.CompilerParams(
            use_tc_tiling_on_sc=False, needs_layout_passes=False
        ),
    )(functools.partial(_emb_body, n_blocks))

    out = emb(w2, idx2, hidx)
    return out.reshape(batch, seq, D_MODEL)


# speedup vs baseline: 1.7723x; 1.7723x over previous
import functools

import jax
import jax.numpy as jnp
from jax import lax
from jax.experimental import pallas as pl
from jax.experimental.pallas import tpu as pltpu
from jax.experimental.pallas import tpu_sc as plsc

D_MODEL = 32
NUM_WORKERS = 32
BLOCK = 256
HI_ROWS = 1024
PACK = 128 // D_MODEL
TROWS = HI_ROWS // PACK

_DYN_GATHER_DNUMS = lax.GatherDimensionNumbers(
    offset_dims=(), collapsed_slice_dims=(0,), start_index_map=(0,)
)


def _lane_perm(a, idx):
    return lax.gather(
        a, idx[:, None], _DYN_GATHER_DNUMS, slice_sizes=(1,),
        mode=lax.GatherScatterMode.PROMISE_IN_BOUNDS,
    )


def _emb_body(n_blocks, tlo_hbm, thi_hbm, idx_hbm, out_hbm,
              tlo, tlo_s, thi, idx0, idx1, obuf0, obuf1,
              isem, osem):
    cid = lax.axis_index("c")
    sid = lax.axis_index("s")
    wid = sid * 2 + cid
    base4 = wid * (n_blocks * (BLOCK // PACK))
    iota16 = lax.iota(jnp.int32, 16)
    swap_idx = lax.bitwise_xor(iota16, 1)
    odd_idx = lax.bitwise_or(iota16, 1)
    even_idx = lax.bitwise_and(iota16, ~1)
    sgn = jnp.where(lax.bitwise_and(iota16, 1) == 0, 1.0, -1.0).astype(jnp.float32)

    pltpu.sync_copy(tlo_hbm, tlo)
    pltpu.sync_copy(thi_hbm, thi)

    def mk_swapped(rr, carry):
        row = lax.shift_right_logical(rr, 2)
        col = lax.bitwise_and(rr, 3) * D_MODEL
        for h in (0, 16):
            a = tlo[row, pl.ds(col + h, 16)]
            tlo_s[row, pl.ds(col + h, 16)] = _lane_perm(a, swap_idx) * sgn
        return carry

    lax.fori_loop(0, HI_ROWS, mk_swapped, 0)

    def compute_block(idx_v, obuf):
        @plsc.parallel_loop(0, BLOCK // 16, step=1)
        def _(i):
            p16 = idx_v[pl.ds(i * 16, 16)]
            for u in range(16):
                p = p16[u]
                hi = lax.shift_right_logical(p, 10)
                lo = lax.bitwise_and(p, 1023)
                row_lo = lax.shift_right_logical(lo, 2)
                col_lo = lax.bitwise_and(lo, 3) * D_MODEL
                row_hi = lax.shift_right_logical(hi, 2)
                col_hi = lax.bitwise_and(hi, 3) * D_MODEL
                for h in (0, 16):
                    a = tlo[row_lo, pl.ds(col_lo + h, 16)]
                    a_s = tlo_s[row_lo, pl.ds(col_lo + h, 16)]
                    b = thi[row_hi, pl.ds(col_hi + h, 16)]
                    x = _lane_perm(b, odd_idx)
                    v = _lane_perm(b, even_idx)
                    obuf[i * 4 + u // 4, pl.ds((u % 4) * D_MODEL + h, 16)] = (
                        a * x + a_s * v)

    def load_idx(b, idx_v):
        pltpu.async_copy(idx_hbm.at[wid, pl.ds(b * BLOCK, BLOCK)], idx_v, isem)

    def wait_idx(b, idx_v):
        pltpu.make_async_copy(
            idx_hbm.at[wid, pl.ds(b * BLOCK, BLOCK)], idx_v, isem).wait()

    def process(b, idx_v, obuf):
        wait_idx(b, idx_v)

        @pl.when(b >= 2)
        def _():
            pltpu.make_async_copy(
                obuf, out_hbm.at[pl.ds(base4, BLOCK // PACK)], osem).wait()

        compute_block(idx_v, obuf)
        pltpu.async_copy(
            obuf, out_hbm.at[pl.ds(base4 + b * (BLOCK // PACK), BLOCK // PACK)],
            osem)

        @pl.when(b + 2 < n_blocks)
        def _():
            load_idx(b + 2, idx_v)

    load_idx(0, idx0)
    load_idx(1, idx1)

    def body(k, carry):
        process(2 * k, idx0, obuf0)
        process(2 * k + 1, idx1, obuf1)
        return carry

    lax.fori_loop(0, n_blocks // 2, body, 0)
    for obuf in (obuf0, obuf1):
        pltpu.make_async_copy(
            obuf, out_hbm.at[pl.ds(base4, BLOCK // PACK)], osem).wait()


def kernel(x, w):
    batch, seq = x.shape
    n_total = batch * seq
    n_per_worker = n_total // NUM_WORKERS
    n_blocks = n_per_worker // BLOCK
    c_in = w.shape[0]
    idx2 = x.reshape(NUM_WORKERS, n_per_worker)
    tlo_src = w[:HI_ROWS].reshape(TROWS, 128)
    thi_strided = w[::HI_ROWS]
    thi_src = jnp.concatenate(
        [thi_strided,
         jnp.zeros((HI_ROWS - thi_strided.shape[0], D_MODEL), w.dtype)]
    ).reshape(TROWS, 128)

    mesh = plsc.VectorSubcoreMesh(core_axis_name="c", subcore_axis_name="s")
    emb = functools.partial(
        pl.kernel,
        out_type=jax.ShapeDtypeStruct((n_total // PACK, 128), jnp.float32),
        mesh=mesh,
        scratch_types=[
            pltpu.VMEM((TROWS, 128), jnp.float32),
            pltpu.VMEM((TROWS, 128), jnp.float32),
            pltpu.VMEM((TROWS, 128), jnp.float32),
            pltpu.VMEM((BLOCK,), jnp.int32),
            pltpu.VMEM((BLOCK,), jnp.int32),
            pltpu.VMEM((BLOCK // PACK, 128), jnp.float32),
            pltpu.VMEM((BLOCK // PACK, 128), jnp.float32),
            pltpu.SemaphoreType.DMA,
            pltpu.SemaphoreType.DMA,
        ],
        compiler_params=pltpu.CompilerParams(
            use_tc_tiling_on_sc=False, needs_layout_passes=False
        ),
    )(functools.partial(_emb_body, n_blocks))

    out = emb(tlo_src, thi_src, idx2)
    return out.reshape(batch, seq, D_MODEL)
